# Initial kernel scaffold; baseline (speedup 1.0000x reference)
import jax
import jax.numpy as jnp
from jax import lax
from jax.experimental import pallas as pl
from jax.experimental.pallas import tpu as pltpu
from jax.experimental.pallas import tpu_sc as plsc

_EMBED = 64
_NC = 2
_NS = 16
_NW = _NC * _NS
_CHUNK = 128


def _sc_gather(idx_hbm, table_hbm, out_hbm, idx_v, rows0, rows1, sem0, sem1):
    cpw = idx_hbm.shape[0] // _NW
    wid = lax.axis_index("s") * _NC + lax.axis_index("c")
    cbase = wid * cpw
    pltpu.sync_copy(idx_hbm.at[pl.ds(cbase, cpw)], idx_v)

    rows = (rows0, rows1)
    sems = (sem0, sem1)

    def start(j, b):
        pltpu.async_copy(table_hbm.at[idx_v.at[j]], rows[b], sems[b])

    def drain(j, b):
        pltpu.make_async_copy(table_hbm.at[idx_v.at[j]], rows[b], sems[b]).wait()
        pltpu.sync_copy(rows[b], out_hbm.at[pl.ds((cbase + j) * _CHUNK, _CHUNK)])

    start(0, 0)
    start(1, 1)

    def body(step, carry):
        for b in range(2):
            j = step * 2 + b
            drain(j, b)
            start(j + 2, b)
        return carry

    lax.fori_loop(0, cpw // 2 - 1, body, 0)
    drain(cpw - 2, 0)
    drain(cpw - 1, 1)


def kernel(x, encoding):
    n, s = x.shape
    b = n * s
    idx = x.reshape(b // _CHUNK, _CHUNK).astype(jnp.int32)
    cpw = (b // _CHUNK) // _NW
    out = pl.kernel(
        _sc_gather,
        out_type=jax.ShapeDtypeStruct((b, _EMBED), jnp.float32),
        mesh=plsc.VectorSubcoreMesh(core_axis_name="c", subcore_axis_name="s"),
        scratch_types=[
            pltpu.VMEM((cpw, _CHUNK), jnp.int32),
            pltpu.VMEM((_CHUNK, _EMBED), jnp.float32),
            pltpu.VMEM((_CHUNK, _EMBED), jnp.float32),
            pltpu.SemaphoreType.DMA,
            pltpu.SemaphoreType.DMA,
        ],
    )(idx, encoding)
    return out.reshape(n, s, _EMBED)


# speedup vs baseline: 4.7888x; 4.7888x over previous
import jax
import jax.numpy as jnp
from jax import lax
from jax.experimental import pallas as pl
from jax.experimental.pallas import tpu as pltpu
from jax.experimental.pallas import tpu_sc as plsc

_EMBED = 64
_NC = 2
_NS = 16
_NW = _NC * _NS
_CHUNK = 128


def _sc_gather(idx_hbm, table_hbm, out_hbm, idx_v, rows0, rows1, sem0, sem1):
    cpw = idx_hbm.shape[1]
    wid = lax.axis_index("s") * _NC + lax.axis_index("c")
    cbase = wid * cpw
    pltpu.sync_copy(idx_hbm.at[wid], idx_v)

    rows = (rows0, rows1)
    sems = (sem0, sem1)

    def start(j, b):
        pltpu.async_copy(table_hbm.at[idx_v.at[j]], rows[b], sems[b])

    def drain(j, b):
        pltpu.make_async_copy(table_hbm.at[idx_v.at[j]], rows[b], sems[b]).wait()
        pltpu.sync_copy(rows[b], out_hbm.at[pl.ds((cbase + j) * _CHUNK, _CHUNK)])

    start(0, 0)
    start(1, 1)

    def body(step, carry):
        for b in range(2):
            j = step * 2 + b
            drain(j, b)
            start(j + 2, b)
        return carry

    lax.fori_loop(0, cpw // 2 - 1, body, 0)
    drain(cpw - 2, 0)
    drain(cpw - 1, 1)


def kernel(x, encoding):
    n, s = x.shape
    b = n * s
    cpw = (b // _CHUNK) // _NW
    idx = x.reshape(_NW, cpw, _CHUNK).astype(jnp.int32)
    out = pl.kernel(
        _sc_gather,
        out_type=jax.ShapeDtypeStruct((b, _EMBED), jnp.float32),
        mesh=plsc.VectorSubcoreMesh(core_axis_name="c", subcore_axis_name="s"),
        compiler_params=pltpu.CompilerParams(use_tc_tiling_on_sc=False),
        scratch_types=[
            pltpu.VMEM((cpw, _CHUNK), jnp.int32),
            pltpu.VMEM((_CHUNK, _EMBED), jnp.float32),
            pltpu.VMEM((_CHUNK, _EMBED), jnp.float32),
            pltpu.SemaphoreType.DMA,
            pltpu.SemaphoreType.DMA,
        ],
    )(idx, encoding)
    return out.reshape(n, s, _EMBED)


# speedup vs baseline: 4.8263x; 1.0078x over previous
import jax
import jax.numpy as jnp
from jax import lax
from jax.experimental import pallas as pl
from jax.experimental.pallas import tpu as pltpu
from jax.experimental.pallas import tpu_sc as plsc

_EMBED = 64
_NC = 2
_NS = 16
_NW = _NC * _NS
_CHUNK = 128


_NBUF = 5


def _sc_gather(idx_hbm, table_hbm, out_hbm, idx_v, rows, gsem, wsem):
    cpw = idx_hbm.shape[1]
    wid = lax.axis_index("s") * _NC + lax.axis_index("c")
    cbase = wid * cpw
    pltpu.sync_copy(idx_hbm.at[wid], idx_v)

    def gstart(j, b):
        pltpu.async_copy(table_hbm.at[idx_v.at[j]], rows.at[b], gsem.at[b])

    def gwait(j, b):
        pltpu.make_async_copy(
            table_hbm.at[idx_v.at[j]], rows.at[b], gsem.at[b]).wait()

    def out_slice(j):
        return out_hbm.at[pl.ds((cbase + j) * _CHUNK, _CHUNK)]

    def wstart(j, b):
        pltpu.async_copy(rows.at[b], out_slice(j), wsem.at[b])

    def wwait(j, b):
        pltpu.make_async_copy(rows.at[b], out_slice(j), wsem.at[b]).wait()

    for b in range(_NBUF):
        gstart(b, b)

    nsteps = cpw // _NBUF

    def body(step, carry):
        base = step * _NBUF
        for b in range(_NBUF):
            gwait(base + b, b)
            wstart(base + b, b)
        for b in range(_NBUF):
            wwait(base + b, b)
            gstart(base + _NBUF + b, b)
        return carry

    lax.fori_loop(0, nsteps - 1, body, 0)
    tail = (nsteps - 1) * _NBUF
    for b in range(_NBUF):
        gwait(tail + b, b)
        wstart(tail + b, b)
    for b in range(_NBUF):
        wwait(tail + b, b)


def kernel(x, encoding):
    n, s = x.shape
    b = n * s
    cpw = (b // _CHUNK) // _NW
    idx = x.reshape(_NW, cpw, _CHUNK).astype(jnp.int32)
    out = pl.kernel(
        _sc_gather,
        out_type=jax.ShapeDtypeStruct((b, _EMBED), jnp.float32),
        mesh=plsc.VectorSubcoreMesh(core_axis_name="c", subcore_axis_name="s"),
        compiler_params=pltpu.CompilerParams(use_tc_tiling_on_sc=False),
        scratch_types=[
            pltpu.VMEM((cpw, _CHUNK), jnp.int32),
            pltpu.VMEM((_NBUF, _CHUNK, _EMBED), jnp.float32),
            pltpu.SemaphoreType.DMA((_NBUF,)),
            pltpu.SemaphoreType.DMA((_NBUF,)),
        ],
    )(idx, encoding)
    return out.reshape(n, s, _EMBED)
